# Initial kernel scaffold; baseline (speedup 1.0000x reference)
import jax
import jax.numpy as jnp
from jax.experimental import pallas as pl


def kernel(x, edge_index, W1, b1, W2, b2):
    raise NotImplementedError("write your pallas kernel here")


# speedup vs baseline: 2.3581x; 2.3581x over previous
import jax
import jax.numpy as jnp
from jax import lax
from jax.experimental import pallas as pl
from jax.experimental.pallas import tpu as pltpu
import jax.experimental.pallas.tpu_sc as plsc

N = 10000
E = 320000
F_IN = 128
F_HID = 128
F_OUT = 40
F_PAD = 128

NC = 2
NS = 16
NW = NC * NS
CHUNK = 128

HALF = 5120
ACC_ROWS = HALF + 128
ACC_PER_TILE = ACC_ROWS // NS
EPT_PART = E // NW
BKT_CAP = 5632
BKT_CHUNKS = BKT_CAP // CHUNK

N_PAD = 10112
E_PAD = 323584
EPT_32 = E_PAD // NW
DEG_ROWS_PER_TILE = N_PAD // NS

BM = 1000


def _part_body(src_hbm, dst_hbm, sbkt_hbm, dbkt_hbm, srcv, dstv, bs, bd):
    c = lax.axis_index("c")
    s = lax.axis_index("s")
    w = c * NS + s
    base = w * EPT_PART

    pltpu.sync_copy(src_hbm.at[pl.ds(base, EPT_PART)], srcv)
    pltpu.sync_copy(dst_hbm.at[pl.ds(base, EPT_PART)], dstv)

    def _fill(i, _):
        bs[pl.ds(i * 16, 16)] = jnp.zeros((16,), jnp.int32)
        bd[pl.ds(i * 16, 16)] = jnp.full((16,), HALF, jnp.int32)
        return 0
    lax.fori_loop(0, 2 * BKT_CAP // 16, _fill, 0)

    def _group(g, carry):
        offA, offB = carry
        lane1 = lax.iota(jnp.int32, 16) + 1
        s16 = srcv[pl.ds(g * 16, 16)]
        d16 = dstv[pl.ds(g * 16, 16)]
        mA = d16 < HALF
        cumA = plsc.cumsum(mA.astype(jnp.int32))
        pos = jnp.where(mA, offA - 1 + cumA,
                        BKT_CAP + offB - 1 + (lane1 - cumA))
        dloc = jnp.where(mA, d16, d16 - HALF)
        plsc.store_scatter(bs, [pos], s16)
        plsc.store_scatter(bd, [pos], dloc)
        nA = cumA[15]
        offA = jnp.minimum(offA + nA, BKT_CAP - 16)
        offB = jnp.minimum(offB + (16 - nA), BKT_CAP - 16)
        return offA, offB
    lax.fori_loop(0, EPT_PART // 16, _group, (jnp.int32(0), jnp.int32(0)))

    pltpu.sync_copy(bs, sbkt_hbm.at[pl.ds(w * 2 * BKT_CAP, 2 * BKT_CAP)])
    pltpu.sync_copy(bd, dbkt_hbm.at[pl.ds(w * 2 * BKT_CAP, 2 * BKT_CAP)])


def _partition(src, dst):
    mesh = plsc.VectorSubcoreMesh(core_axis_name="c", subcore_axis_name="s")
    return pl.kernel(
        _part_body,
        compiler_params=pltpu.CompilerParams(needs_layout_passes=False),
        out_type=[
            jax.ShapeDtypeStruct((NC * NW * BKT_CAP,), jnp.int32),
            jax.ShapeDtypeStruct((NC * NW * BKT_CAP,), jnp.int32),
        ],
        mesh=mesh,
        scratch_types=[
            pltpu.VMEM((EPT_PART,), jnp.int32),
            pltpu.VMEM((EPT_PART,), jnp.int32),
            pltpu.VMEM((2 * BKT_CAP,), jnp.int32),
            pltpu.VMEM((2 * BKT_CAP,), jnp.int32),
        ],
    )(src, dst)


def _deg_body(dst_hbm, out_hbm, didx, ones_v, zbuf, acc):
    c = lax.axis_index("c")
    s = lax.axis_index("s")
    base = (c * NS + s) * EPT_32

    def _fill(i, _):
        zbuf[pl.ds(i * 16, 16)] = jnp.zeros((16,), jnp.float32)
        return 0
    lax.fori_loop(0, DEG_ROWS_PER_TILE // 16, _fill, 0)
    for k in range(CHUNK // 16):
        ones_v[pl.ds(k * 16, 16)] = jnp.ones((16,), jnp.float32)

    r0 = s * DEG_ROWS_PER_TILE
    pltpu.sync_copy(zbuf, acc.at[pl.ds(r0, DEG_ROWS_PER_TILE)])
    plsc.subcore_barrier()

    def _chunk(j, _):
        pltpu.sync_copy(dst_hbm.at[pl.ds(base + j * CHUNK, CHUNK)], didx)
        pltpu.sync_copy(ones_v, acc.at[didx], add=True)
        return 0
    lax.fori_loop(0, EPT_32 // CHUNK, _chunk, 0)
    plsc.subcore_barrier()

    pltpu.sync_copy(acc.at[pl.ds(r0, DEG_ROWS_PER_TILE)], zbuf)
    pltpu.sync_copy(zbuf, out_hbm.at[pl.ds(c * N_PAD + r0, DEG_ROWS_PER_TILE)])


def _deg_partials(dst_pad):
    mesh = plsc.VectorSubcoreMesh(core_axis_name="c", subcore_axis_name="s")
    return pl.kernel(
        _deg_body,
        out_type=jax.ShapeDtypeStruct((NC * N_PAD,), jnp.float32),
        mesh=mesh,
        scratch_types=[
            pltpu.VMEM((CHUNK,), jnp.int32),
            pltpu.VMEM((CHUNK,), jnp.float32),
            pltpu.VMEM((DEG_ROWS_PER_TILE,), jnp.float32),
            pltpu.VMEM_SHARED((N_PAD,), jnp.float32),
        ],
    )(dst_pad)


def _spmm_body(tab_hbm, sbkt_hbm, dbkt_hbm, out_hbm, sslab, dslab, msg,
               acc, sem):
    c = lax.axis_index("c")
    s = lax.axis_index("s")

    def _fill(i, _):
        for k in range(F_PAD // 16):
            msg[i, pl.ds(k * 16, 16)] = jnp.zeros((16,), jnp.float32)
        return 0
    lax.fori_loop(0, CHUNK, _fill, 0)
    r0 = s * ACC_PER_TILE
    pltpu.sync_copy(msg, acc.at[pl.ds(r0, CHUNK), :])
    pltpu.sync_copy(msg, acc.at[pl.ds(r0 + CHUNK, CHUNK), :])
    pltpu.sync_copy(msg.at[pl.ds(0, ACC_PER_TILE - 2 * CHUNK), :],
                    acc.at[pl.ds(r0 + 2 * CHUNK, ACC_PER_TILE - 2 * CHUNK), :])
    plsc.subcore_barrier()

    for b in range(2):
        w = 2 * s + b
        base = (w * 2 + c) * BKT_CAP
        pltpu.sync_copy(sbkt_hbm.at[pl.ds(base, BKT_CAP)], sslab)

        def _stage(j, _):
            pltpu.sync_copy(dbkt_hbm.at[pl.ds(base + j * CHUNK, CHUNK)],
                            dslab.at[j])
            return 0
        lax.fori_loop(0, BKT_CHUNKS, _stage, 0)

        def _chunk(j, _):
            pltpu.async_copy(
                tab_hbm.at[sslab.at[pl.ds(j * CHUNK, CHUNK)]], msg, sem
            ).wait()
            pltpu.sync_copy(msg, acc.at[dslab.at[j]], add=True)
            return 0
        lax.fori_loop(0, BKT_CHUNKS, _chunk, 0)
    plsc.subcore_barrier()

    for (rr, nr) in ((r0, CHUNK), (r0 + CHUNK, CHUNK),
                     (r0 + 2 * CHUNK, ACC_PER_TILE - 2 * CHUNK)):
        pltpu.sync_copy(acc.at[pl.ds(rr, nr), :], msg.at[pl.ds(0, nr), :])
        pltpu.sync_copy(msg.at[pl.ds(0, nr), :],
                        out_hbm.at[c, pl.ds(rr, nr), :])


def _spmm(tab, sbkt, dbkt):
    mesh = plsc.VectorSubcoreMesh(core_axis_name="c", subcore_axis_name="s")
    return pl.kernel(
        _spmm_body,
        out_type=jax.ShapeDtypeStruct((NC, ACC_ROWS, F_PAD), jnp.float32),
        mesh=mesh,
        scratch_types=[
            pltpu.VMEM((BKT_CAP,), jnp.int32),
            pltpu.VMEM((BKT_CHUNKS, CHUNK), jnp.int32),
            pltpu.VMEM((CHUNK, F_PAD), jnp.float32),
            pltpu.VMEM_SHARED((ACC_ROWS, F_PAD), jnp.float32),
            pltpu.SemaphoreType.DMA,
        ],
    )(tab, sbkt, dbkt)


def _dinv_block(deg_ref):
    d = deg_ref[0] + deg_ref[1]
    return lax.rsqrt(jnp.maximum(d, 1.0))


def _z1_body(deg_ref, x_ref, w_ref, o_ref):
    dinv = _dinv_block(deg_ref)
    o_ref[...] = jnp.dot(x_ref[...] * dinv, w_ref[...],
                         preferred_element_type=jnp.float32)


def _z1(deg, x, W1):
    return pl.pallas_call(
        _z1_body,
        grid=(N // BM,),
        in_specs=[
            pl.BlockSpec((NC, BM, 1), lambda i: (0, i, 0)),
            pl.BlockSpec((BM, F_IN), lambda i: (i, 0)),
            pl.BlockSpec((F_IN, F_HID), lambda i: (0, 0)),
        ],
        out_specs=pl.BlockSpec((BM, F_HID), lambda i: (i, 0)),
        out_shape=jax.ShapeDtypeStruct((N, F_HID), jnp.float32),
    )(deg, x, W1)


def _z2_body(deg_ref, s1_ref, b1_ref, w2_ref, o_ref):
    dinv = _dinv_block(deg_ref)
    h = s1_ref[...] * dinv + b1_ref[...]
    h = jnp.maximum(h, 0.0) * dinv
    o_ref[...] = jnp.dot(h, w2_ref[...], preferred_element_type=jnp.float32)


def _z2(deg, s1, b1r, W2p):
    return pl.pallas_call(
        _z2_body,
        grid=(N // BM,),
        in_specs=[
            pl.BlockSpec((NC, BM, 1), lambda i: (0, i, 0)),
            pl.BlockSpec((BM, F_HID), lambda i: (i, 0)),
            pl.BlockSpec((1, F_HID), lambda i: (0, 0)),
            pl.BlockSpec((F_HID, F_PAD), lambda i: (0, 0)),
        ],
        out_specs=pl.BlockSpec((BM, F_PAD), lambda i: (i, 0)),
        out_shape=jax.ShapeDtypeStruct((N, F_PAD), jnp.float32),
    )(deg, s1, b1r, W2p)


def _out_body(deg_ref, s2_ref, b2_ref, o_ref):
    dinv = _dinv_block(deg_ref)
    o = s2_ref[...] * dinv + b2_ref[...]
    col = lax.broadcasted_iota(jnp.int32, (BM, F_PAD), 1)
    valid = col < F_OUT
    om = jnp.where(valid, o, -jnp.inf)
    m = jnp.max(om, axis=1, keepdims=True)
    e = jnp.where(valid, jnp.exp(o - m), 0.0)
    lse = jnp.log(jnp.sum(e, axis=1, keepdims=True))
    o_ref[...] = o - m - lse


def _logsoftmax_out(deg, s2, b2r):
    return pl.pallas_call(
        _out_body,
        grid=(N // BM,),
        in_specs=[
            pl.BlockSpec((NC, BM, 1), lambda i: (0, i, 0)),
            pl.BlockSpec((BM, F_PAD), lambda i: (i, 0)),
            pl.BlockSpec((1, F_PAD), lambda i: (0, 0)),
        ],
        out_specs=pl.BlockSpec((BM, F_PAD), lambda i: (i, 0)),
        out_shape=jax.ShapeDtypeStruct((N, F_PAD), jnp.float32),
    )(deg, s2, b2r)


def _assemble(s):
    return jnp.concatenate([s[0, :HALF, :], s[1, :N - HALF, :]], axis=0)


def kernel(x, edge_index, W1, b1, W2, b2):
    src = edge_index[0]
    dst = edge_index[1]
    pad = E_PAD - E
    dst_pad = jnp.concatenate([dst, jnp.full((pad,), N, jnp.int32)])

    sbkt, dbkt = _partition(src, dst)
    deg = _deg_partials(dst_pad)
    deg3 = deg.reshape(NC, N_PAD, 1)

    z1 = _z1(deg3, x, W1)
    s1 = _assemble(_spmm(z1, sbkt, dbkt))

    b1r = b1.reshape(1, F_HID)
    W2p = jnp.pad(W2, ((0, 0), (0, F_PAD - F_OUT)))
    z2 = _z2(deg3, s1, b1r, W2p)
    s2 = _assemble(_spmm(z2, sbkt, dbkt))

    b2r = jnp.pad(b2, (0, F_PAD - F_OUT)).reshape(1, F_PAD)
    out = _logsoftmax_out(deg3, s2, b2r)
    return out[:, :F_OUT]


# speedup vs baseline: 2.3984x; 1.0171x over previous
import jax
import jax.numpy as jnp
from jax import lax
from jax.experimental import pallas as pl
from jax.experimental.pallas import tpu as pltpu
import jax.experimental.pallas.tpu_sc as plsc

N = 10000
E = 320000
F_IN = 128
F_HID = 128
F_OUT = 40
F_PAD = 128

NC = 2
NS = 16
NW = NC * NS
CHUNK = 128

HALF = 5120
ACC_ROWS = HALF + 128
ACC_PER_TILE = ACC_ROWS // NS
EPT_PART = E // NW
BKT_CAP = 5632
BKT_CHUNKS = BKT_CAP // CHUNK
SLAB = BKT_CAP // 2
SLAB_CHUNKS = SLAB // CHUNK

N_PAD = 10112
E_PAD = 323584
EPT_32 = E_PAD // NW
DEG_ROWS_PER_TILE = N_PAD // NS

BM = 1000


def _part_body(src_hbm, dst_hbm, sbkt_hbm, dbkt_hbm, srcv, dstv, bs, bd):
    c = lax.axis_index("c")
    s = lax.axis_index("s")
    w = c * NS + s
    base = w * EPT_PART

    pltpu.sync_copy(src_hbm.at[pl.ds(base, EPT_PART)], srcv)
    pltpu.sync_copy(dst_hbm.at[pl.ds(base, EPT_PART)], dstv)

    def _fill(i, _):
        bs[pl.ds(i * 16, 16)] = jnp.zeros((16,), jnp.int32)
        bd[pl.ds(i * 16, 16)] = jnp.full((16,), HALF, jnp.int32)
        return 0
    lax.fori_loop(0, 2 * BKT_CAP // 16, _fill, 0)

    def _group(g, carry):
        offA, offB = carry
        lane1 = lax.iota(jnp.int32, 16) + 1
        s16 = srcv[pl.ds(g * 16, 16)]
        d16 = dstv[pl.ds(g * 16, 16)]
        mA = d16 < HALF
        cumA = plsc.cumsum(mA.astype(jnp.int32))
        pos = jnp.where(mA, offA - 1 + cumA,
                        BKT_CAP + offB - 1 + (lane1 - cumA))
        dloc = jnp.where(mA, d16, d16 - HALF)
        plsc.store_scatter(bs, [pos], s16)
        plsc.store_scatter(bd, [pos], dloc)
        nA = cumA[15]
        offA = jnp.minimum(offA + nA, BKT_CAP - 16)
        offB = jnp.minimum(offB + (16 - nA), BKT_CAP - 16)
        return offA, offB
    lax.fori_loop(0, EPT_PART // 16, _group, (jnp.int32(0), jnp.int32(0)))

    pltpu.sync_copy(bs, sbkt_hbm.at[pl.ds(w * 2 * BKT_CAP, 2 * BKT_CAP)])
    pltpu.sync_copy(bd, dbkt_hbm.at[pl.ds(w * 2 * BKT_CAP, 2 * BKT_CAP)])


def _partition(src, dst):
    mesh = plsc.VectorSubcoreMesh(core_axis_name="c", subcore_axis_name="s")
    return pl.kernel(
        _part_body,
        compiler_params=pltpu.CompilerParams(needs_layout_passes=False),
        out_type=[
            jax.ShapeDtypeStruct((NC * NW * BKT_CAP,), jnp.int32),
            jax.ShapeDtypeStruct((NC * NW * BKT_CAP,), jnp.int32),
        ],
        mesh=mesh,
        scratch_types=[
            pltpu.VMEM((EPT_PART,), jnp.int32),
            pltpu.VMEM((EPT_PART,), jnp.int32),
            pltpu.VMEM((2 * BKT_CAP,), jnp.int32),
            pltpu.VMEM((2 * BKT_CAP,), jnp.int32),
        ],
    )(src, dst)


def _deg_body(dst_hbm, out_hbm, didx, ones_v, zbuf, acc):
    c = lax.axis_index("c")
    s = lax.axis_index("s")
    base = (c * NS + s) * EPT_32

    def _fill(i, _):
        zbuf[pl.ds(i * 16, 16)] = jnp.zeros((16,), jnp.float32)
        return 0
    lax.fori_loop(0, DEG_ROWS_PER_TILE // 16, _fill, 0)
    for k in range(CHUNK // 16):
        ones_v[pl.ds(k * 16, 16)] = jnp.ones((16,), jnp.float32)

    r0 = s * DEG_ROWS_PER_TILE
    pltpu.sync_copy(zbuf, acc.at[pl.ds(r0, DEG_ROWS_PER_TILE)])
    plsc.subcore_barrier()

    def _chunk(j, _):
        pltpu.sync_copy(dst_hbm.at[pl.ds(base + j * CHUNK, CHUNK)], didx)
        pltpu.sync_copy(ones_v, acc.at[didx], add=True)
        return 0
    lax.fori_loop(0, EPT_32 // CHUNK, _chunk, 0)
    plsc.subcore_barrier()

    pltpu.sync_copy(acc.at[pl.ds(r0, DEG_ROWS_PER_TILE)], zbuf)
    pltpu.sync_copy(zbuf, out_hbm.at[pl.ds(c * N_PAD + r0, DEG_ROWS_PER_TILE)])


def _deg_partials(dst_pad):
    mesh = plsc.VectorSubcoreMesh(core_axis_name="c", subcore_axis_name="s")
    return pl.kernel(
        _deg_body,
        out_type=jax.ShapeDtypeStruct((NC * N_PAD,), jnp.float32),
        mesh=mesh,
        scratch_types=[
            pltpu.VMEM((CHUNK,), jnp.int32),
            pltpu.VMEM((CHUNK,), jnp.float32),
            pltpu.VMEM((DEG_ROWS_PER_TILE,), jnp.float32),
            pltpu.VMEM_SHARED((N_PAD,), jnp.float32),
        ],
    )(dst_pad)


def _spmm_body(tab_hbm, sbkt_hbm, dbkt_hbm, out_hbm, sslab, dslab, msg,
               acc, sem):
    c = lax.axis_index("c")
    s = lax.axis_index("s")

    def _fill(i, _):
        for k in range(F_PAD // 16):
            msg[0, i, pl.ds(k * 16, 16)] = jnp.zeros((16,), jnp.float32)
        return 0
    lax.fori_loop(0, CHUNK, _fill, 0)
    r0 = s * ACC_PER_TILE
    pltpu.sync_copy(msg.at[0], acc.at[pl.ds(r0, CHUNK), :])
    pltpu.sync_copy(msg.at[0], acc.at[pl.ds(r0 + CHUNK, CHUNK), :])
    pltpu.sync_copy(msg.at[0, pl.ds(0, ACC_PER_TILE - 2 * CHUNK), :],
                    acc.at[pl.ds(r0 + 2 * CHUNK, ACC_PER_TILE - 2 * CHUNK), :])
    plsc.subcore_barrier()

    for b in range(2):
        w = 2 * s + b
        for p in range(2):
            base = (w * 2 + c) * BKT_CAP + p * SLAB
            pltpu.sync_copy(sbkt_hbm.at[pl.ds(base, SLAB)], sslab)

            def _stage(j, _):
                pltpu.sync_copy(dbkt_hbm.at[pl.ds(base + j * CHUNK, CHUNK)],
                                dslab.at[j])
                return 0
            lax.fori_loop(0, SLAB_CHUNKS, _stage, 0)

            pltpu.async_copy(
                tab_hbm.at[sslab.at[pl.ds(0, CHUNK)]], msg.at[0], sem)

            def _chunk(j, _):
                nxt = j + 1

                @pl.when(nxt < SLAB_CHUNKS)
                def _():
                    pltpu.async_copy(
                        tab_hbm.at[sslab.at[pl.ds(nxt * CHUNK, CHUNK)]],
                        msg.at[lax.rem(nxt, 2)], sem)
                cur = lax.rem(j, 2)
                pltpu.make_async_copy(
                    tab_hbm.at[sslab.at[pl.ds(0, CHUNK)]],
                    msg.at[cur], sem).wait()
                pltpu.sync_copy(msg.at[cur], acc.at[dslab.at[j]], add=True)
                return 0
            lax.fori_loop(0, SLAB_CHUNKS, _chunk, 0)
    plsc.subcore_barrier()

    for (rr, nr) in ((r0, CHUNK), (r0 + CHUNK, CHUNK),
                     (r0 + 2 * CHUNK, ACC_PER_TILE - 2 * CHUNK)):
        pltpu.sync_copy(acc.at[pl.ds(rr, nr), :], msg.at[0, pl.ds(0, nr), :])
        pltpu.sync_copy(msg.at[0, pl.ds(0, nr), :],
                        out_hbm.at[c, pl.ds(rr, nr), :])


def _spmm(tab, sbkt, dbkt):
    mesh = plsc.VectorSubcoreMesh(core_axis_name="c", subcore_axis_name="s")
    return pl.kernel(
        _spmm_body,
        out_type=jax.ShapeDtypeStruct((NC, ACC_ROWS, F_PAD), jnp.float32),
        mesh=mesh,
        scratch_types=[
            pltpu.VMEM((SLAB,), jnp.int32),
            pltpu.VMEM((SLAB_CHUNKS, CHUNK), jnp.int32),
            pltpu.VMEM((2, CHUNK, F_PAD), jnp.float32),
            pltpu.VMEM_SHARED((ACC_ROWS, F_PAD), jnp.float32),
            pltpu.SemaphoreType.DMA,
        ],
    )(tab, sbkt, dbkt)


def _dinv_block(deg_ref):
    d = deg_ref[0] + deg_ref[1]
    return lax.rsqrt(jnp.maximum(d, 1.0))


def _z1_body(deg_ref, x_ref, w_ref, o_ref):
    dinv = _dinv_block(deg_ref)
    o_ref[...] = jnp.dot(x_ref[...] * dinv, w_ref[...],
                         preferred_element_type=jnp.float32)


def _z1(deg, x, W1):
    return pl.pallas_call(
        _z1_body,
        grid=(N // BM,),
        in_specs=[
            pl.BlockSpec((NC, BM, 1), lambda i: (0, i, 0)),
            pl.BlockSpec((BM, F_IN), lambda i: (i, 0)),
            pl.BlockSpec((F_IN, F_HID), lambda i: (0, 0)),
        ],
        out_specs=pl.BlockSpec((BM, F_HID), lambda i: (i, 0)),
        out_shape=jax.ShapeDtypeStruct((N, F_HID), jnp.float32),
    )(deg, x, W1)


def _z2_body(deg_ref, s1_ref, b1_ref, w2_ref, o_ref):
    dinv = _dinv_block(deg_ref)
    h = s1_ref[...] * dinv + b1_ref[...]
    h = jnp.maximum(h, 0.0) * dinv
    o_ref[...] = jnp.dot(h, w2_ref[...], preferred_element_type=jnp.float32)


def _z2(deg, s1, b1r, W2p):
    return pl.pallas_call(
        _z2_body,
        grid=(N // BM,),
        in_specs=[
            pl.BlockSpec((NC, BM, 1), lambda i: (0, i, 0)),
            pl.BlockSpec((BM, F_HID), lambda i: (i, 0)),
            pl.BlockSpec((1, F_HID), lambda i: (0, 0)),
            pl.BlockSpec((F_HID, F_PAD), lambda i: (0, 0)),
        ],
        out_specs=pl.BlockSpec((BM, F_PAD), lambda i: (i, 0)),
        out_shape=jax.ShapeDtypeStruct((N, F_PAD), jnp.float32),
    )(deg, s1, b1r, W2p)


def _out_body(deg_ref, s2_ref, b2_ref, o_ref):
    dinv = _dinv_block(deg_ref)
    o = s2_ref[...] * dinv + b2_ref[...]
    col = lax.broadcasted_iota(jnp.int32, (BM, F_PAD), 1)
    valid = col < F_OUT
    om = jnp.where(valid, o, -jnp.inf)
    m = jnp.max(om, axis=1, keepdims=True)
    e = jnp.where(valid, jnp.exp(o - m), 0.0)
    lse = jnp.log(jnp.sum(e, axis=1, keepdims=True))
    o_ref[...] = o - m - lse


def _logsoftmax_out(deg, s2, b2r):
    return pl.pallas_call(
        _out_body,
        grid=(N // BM,),
        in_specs=[
            pl.BlockSpec((NC, BM, 1), lambda i: (0, i, 0)),
            pl.BlockSpec((BM, F_PAD), lambda i: (i, 0)),
            pl.BlockSpec((1, F_PAD), lambda i: (0, 0)),
        ],
        out_specs=pl.BlockSpec((BM, F_PAD), lambda i: (i, 0)),
        out_shape=jax.ShapeDtypeStruct((N, F_PAD), jnp.float32),
    )(deg, s2, b2r)


def _assemble(s):
    return jnp.concatenate([s[0, :HALF, :], s[1, :N - HALF, :]], axis=0)


def kernel(x, edge_index, W1, b1, W2, b2):
    src = edge_index[0]
    dst = edge_index[1]
    pad = E_PAD - E
    dst_pad = jnp.concatenate([dst, jnp.full((pad,), N, jnp.int32)])

    sbkt, dbkt = _partition(src, dst)
    deg = _deg_partials(dst_pad)
    deg3 = deg.reshape(NC, N_PAD, 1)

    z1 = _z1(deg3, x, W1)
    s1 = _assemble(_spmm(z1, sbkt, dbkt))

    b1r = b1.reshape(1, F_HID)
    W2p = jnp.pad(W2, ((0, 0), (0, F_PAD - F_OUT)))
    z2 = _z2(deg3, s1, b1r, W2p)
    s2 = _assemble(_spmm(z2, sbkt, dbkt))

    b2r = jnp.pad(b2, (0, F_PAD - F_OUT)).reshape(1, F_PAD)
    out = _logsoftmax_out(deg3, s2, b2r)
    return out[:, :F_OUT]
